# baseline (speedup 1.0000x reference)
import functools

import jax
import jax.numpy as jnp
from jax import lax
from jax.experimental import pallas as pl
from jax.experimental.pallas import tpu as pltpu
from jax.experimental.pallas import tpu_sc as plsc

VOCAB = 1000000
EMBED_DIM = 64
BATCH = 4096


def _sc_gather_pair(target, context, in_embed_weight, out_embed_weight):
    info = plsc.get_sparse_core_info()
    nw = info.num_cores * info.num_subcores
    b_per_w = BATCH // nw
    mesh = plsc.VectorSubcoreMesh(core_axis_name="c", subcore_axis_name="s")

    @functools.partial(
        pl.kernel,
        out_type=(
            jax.ShapeDtypeStruct((BATCH, EMBED_DIM), jnp.float32),
            jax.ShapeDtypeStruct((BATCH, EMBED_DIM), jnp.float32),
        ),
        mesh=mesh,
        compiler_params=pltpu.CompilerParams(use_tc_tiling_on_sc=False),
        scratch_types=[
            pltpu.VMEM((b_per_w,), jnp.int32),
            pltpu.VMEM((b_per_w,), jnp.int32),
            pltpu.VMEM((b_per_w, EMBED_DIM), jnp.float32),
            pltpu.VMEM((b_per_w, EMBED_DIM), jnp.float32),
            pltpu.SemaphoreType.DMA,
            pltpu.SemaphoreType.DMA,
        ],
    )
    def gather_kernel(tgt_hbm, ctx_hbm, in_tab, out_tab, tgt_rows_hbm,
                      ctx_rows_hbm, idx_t, idx_c, rows_t, rows_c, sem_t, sem_c):
        wid = lax.axis_index("s") * info.num_cores + lax.axis_index("c")
        base = wid * b_per_w
        pltpu.sync_copy(tgt_hbm.at[pl.ds(base, b_per_w)], idx_t)
        pltpu.sync_copy(ctx_hbm.at[pl.ds(base, b_per_w)], idx_c)
        cp_t = pltpu.async_copy(in_tab.at[idx_t], rows_t, sem_t)
        cp_c = pltpu.async_copy(out_tab.at[idx_c], rows_c, sem_c)
        cp_t.wait()
        cp_c.wait()
        pltpu.sync_copy(rows_t, tgt_rows_hbm.at[pl.ds(base, b_per_w)])
        pltpu.sync_copy(rows_c, ctx_rows_hbm.at[pl.ds(base, b_per_w)])

    return gather_kernel(target, context, in_embed_weight, out_embed_weight)


def _scores_matmul(tgt_rows, ctx_rows):
    bm = 256

    def mm(a_ref, b_ref, o_ref):
        o_ref[...] = lax.dot_general(
            a_ref[...], b_ref[...],
            dimension_numbers=(((1,), (1,)), ((), ())),
            preferred_element_type=jnp.float32,
        )

    return pl.pallas_call(
        mm,
        grid=(BATCH // bm,),
        in_specs=[
            pl.BlockSpec((bm, EMBED_DIM), lambda i: (i, 0)),
            pl.BlockSpec((BATCH, EMBED_DIM), lambda i: (0, 0)),
        ],
        out_specs=pl.BlockSpec((bm, BATCH), lambda i: (i, 0)),
        out_shape=jax.ShapeDtypeStruct((BATCH, BATCH), jnp.float32),
    )(tgt_rows, ctx_rows)


def kernel(target, context, in_embed_weight, out_embed_weight):
    tgt_rows, ctx_rows = _sc_gather_pair(
        target, context, in_embed_weight, out_embed_weight)
    return _scores_matmul(tgt_rows, ctx_rows)


# speedup vs baseline: 1.3520x; 1.3520x over previous
import functools

import jax
import jax.numpy as jnp
from jax import lax
from jax.experimental import pallas as pl
from jax.experimental.pallas import tpu as pltpu
from jax.experimental.pallas import tpu_sc as plsc

VOCAB = 1000000
EMBED_DIM = 64
BATCH = 4096


def _sc_gather_pair(target, context, in_tab, out_tab):
    info = plsc.get_sparse_core_info()
    nw = info.num_cores * info.num_subcores
    bw = BATCH // nw
    mesh = plsc.VectorSubcoreMesh(core_axis_name="c", subcore_axis_name="s")

    @functools.partial(
        pl.kernel,
        out_type=(
            jax.ShapeDtypeStruct((BATCH, EMBED_DIM), jnp.float32),
            jax.ShapeDtypeStruct((BATCH, EMBED_DIM), jnp.float32),
        ),
        mesh=mesh,
        compiler_params=pltpu.CompilerParams(needs_layout_passes=False),
        scratch_types=[
            pltpu.VMEM((bw,), jnp.int32),
            pltpu.VMEM((bw, EMBED_DIM), jnp.float32),
            pltpu.SemaphoreType.DMA,
        ],
    )
    def gather_kernel(tgt_hbm, ctx_hbm, in_tab_hbm, out_tab_hbm, tgt_rows_hbm,
                      ctx_rows_hbm, idx_v, rows_v, sem):
        wid = lax.axis_index("s") * info.num_cores + lax.axis_index("c")
        base = wid * bw
        iota = lax.iota(jnp.int32, 16)

        def one_table(idx_hbm, tab, rows_out_hbm):
            pltpu.sync_copy(idx_hbm.at[pl.ds(base, bw)], idx_v)

            for g in range(bw // 16):
                idxg = idx_v[pl.ds(g * 16, 16)]

                def lane_body(l, carry, idxg=idxg, g=g):
                    ri = jnp.sum(jnp.where(iota == l, idxg, 0))
                    pltpu.async_copy(
                        tab.at[pl.ds(ri, 1)],
                        rows_v.at[pl.ds(g * 16 + l, 1)], sem).wait()
                    return carry

                lax.fori_loop(0, 16, lane_body, 0)
            pltpu.sync_copy(rows_v, rows_out_hbm.at[pl.ds(base, bw)])

        one_table(tgt_hbm, in_tab_hbm, tgt_rows_hbm)
        one_table(ctx_hbm, out_tab_hbm, ctx_rows_hbm)

    return gather_kernel(target, context, in_tab, out_tab)


def _scores_matmul(tgt_rows, ctx_rows):
    bm = 256

    def mm(a_ref, b_ref, o_ref):
        o_ref[...] = lax.dot_general(
            a_ref[...], b_ref[...],
            dimension_numbers=(((1,), (1,)), ((), ())),
            preferred_element_type=jnp.float32,
        )

    return pl.pallas_call(
        mm,
        grid=(BATCH // bm,),
        in_specs=[
            pl.BlockSpec((bm, EMBED_DIM), lambda i: (i, 0)),
            pl.BlockSpec((BATCH, EMBED_DIM), lambda i: (0, 0)),
        ],
        out_specs=pl.BlockSpec((bm, BATCH), lambda i: (i, 0)),
        out_shape=jax.ShapeDtypeStruct((BATCH, BATCH), jnp.float32),
    )(tgt_rows, ctx_rows)


def kernel(target, context, in_embed_weight, out_embed_weight):
    tgt_rows, ctx_rows = _sc_gather_pair(
        target, context, in_embed_weight, out_embed_weight)
    return _scores_matmul(tgt_rows, ctx_rows)
